# baseline (speedup 1.0000x reference)
import functools

import jax
import jax.numpy as jnp
import numpy as np
from jax import lax
from jax.experimental import pallas as pl
from jax.experimental.pallas import tpu as pltpu
from jax.experimental.pallas import tpu_sc as plsc

_NB, _NA, _NG, _NC = 16, 3, 52, 80
_NCH = _NA * (_NC + 5)
_SPATIAL = _NG * _NG
_STRIDE_B = _NCH * _SPATIAL
_TOTAL_CELLS = float(_NB * _NA * _SPATIAL)
_NT = 64
_CPAD = 256
_NW = 32
_PER_W = _NT * _CPAD // _NW
_AW = np.array([10.0, 16.0, 33.0], np.float32) / 8.0
_AH = np.array([13.0, 30.0, 23.0], np.float32) / 8.0
_IGNORE = 0.5
_NOOBJ_SCALE = 100.0


def _bce(p, t):
    p = jnp.clip(p, 1e-12, 1.0 - 1e-12)
    return -(t * jnp.log(p) + (1.0 - t) * jnp.log(1.0 - p))


def _target_geometry(t):
    b = t[:, 0].astype(jnp.int32)
    gx = t[:, 2] * _NG
    gy = t[:, 3] * _NG
    gw = t[:, 4] * _NG
    gh = t[:, 5] * _NG
    gi = gx.astype(jnp.int32)
    gj = gy.astype(jnp.int32)
    ious = []
    for a in range(_NA):
        inter = jnp.minimum(_AW[a], gw) * jnp.minimum(_AH[a], gh)
        union = _AW[a] * _AH[a] + 1e-16 + gw * gh - inter
        ious.append(inter / union)
    bn = jnp.where(
        (ious[0] >= ious[1]) & (ious[0] >= ious[2]),
        0,
        jnp.where(ious[1] >= ious[2], 1, 2),
    ).astype(jnp.int32)
    return b, gi, gj, gx, gy, gw, gh, ious, bn


def _idx_body(t_ref, o_ref):
    t = t_ref[...]
    b, gi, gj, *_ = _target_geometry(t)
    base = b * _STRIDE_B + gj * _NG + gi
    c = jnp.minimum(lax.broadcasted_iota(jnp.int32, (_NT, _CPAD), 1), _NCH - 1)
    o_ref[...] = base[:, None] + c * _SPATIAL


_HALF = _SPATIAL // 2


def _sc_gather_body(x_hbm, idx_hbm, out_hbm, conf_hbm, idx_v, rows_v, buf_v, sem):
    wid = lax.axis_index("s") * 2 + lax.axis_index("c")
    base = wid * _PER_W
    pltpu.sync_copy(idx_hbm.at[pl.ds(base, _PER_W)], idx_v)
    pltpu.async_copy(x_hbm.at[idx_v], rows_v, sem).wait()
    pltpu.sync_copy(rows_v, out_hbm.at[pl.ds(base, _PER_W)])
    for r in range(3):
        hc = wid * 3 + r
        b = hc // 6
        a = (hc // 2) % 3
        src = b * _STRIDE_B + (4 + (_NC + 5) * a) * _SPATIAL + (hc % 2) * _HALF
        pltpu.sync_copy(x_hbm.at[pl.ds(src, _HALF)], buf_v)
        pltpu.sync_copy(buf_v, conf_hbm.at[pl.ds(hc * _HALF, _HALF)])


def _losses(g, t, s_dense):
    f32 = jnp.float32
    b, gi, gj, gx, gy, gw, gh, ious, bn = _target_geometry(t)
    labels = t[:, 1].astype(jnp.int32)
    flat_obj = ((b * _NA + bn) * _NG + gj) * _NG + gi

    tidx = lax.broadcasted_iota(jnp.int32, (_NT, _NT), 0)
    oidx = lax.broadcasted_iota(jnp.int32, (_NT, _NT), 1)
    eq = flat_obj[:, None] == flat_obj[None, :]
    winner = ~jnp.any(eq & (oidx > tidx), axis=1)
    wf = winner.astype(f32)
    cnt = jnp.sum(wf)

    b0 = g[:, 0 : _NC + 5]
    b1 = g[:, _NC + 5 : 2 * (_NC + 5)]
    b2 = g[:, 2 * (_NC + 5) : 3 * (_NC + 5)]
    bnf = bn[:, None]
    sel = jnp.where(bnf == 0, b0, jnp.where(bnf == 1, b1, b2))
    px = jax.nn.sigmoid(sel[:, 0])
    py = jax.nn.sigmoid(sel[:, 1])
    pw = sel[:, 2]
    ph = sel[:, 3]
    pconf = jax.nn.sigmoid(sel[:, 4])
    aw = jnp.where(bn == 0, _AW[0], jnp.where(bn == 1, _AW[1], _AW[2]))
    ah = jnp.where(bn == 0, _AH[0], jnp.where(bn == 1, _AH[1], _AH[2]))

    tx = gx - jnp.floor(gx)
    ty = gy - jnp.floor(gy)
    tw = jnp.log(gw / aw + 1e-16)
    th = jnp.log(gh / ah + 1e-16)
    loss_x = jnp.sum(wf * (px - tx) ** 2) / cnt
    loss_y = jnp.sum(wf * (py - ty) ** 2) / cnt
    loss_w = jnp.sum(wf * (pw - tw) ** 2) / cnt
    loss_h = jnp.sum(wf * (ph - th) ** 2) / cnt
    loss_conf_obj = jnp.sum(wf * _bce(pconf, 1.0)) / cnt

    cls_p = jax.nn.sigmoid(sel[:, 5:])
    cidx = lax.broadcasted_iota(jnp.int32, (_NT, _NC), 1)
    onehot = (cidx == labels[:, None]).astype(f32)
    loss_cls = jnp.sum(wf[:, None] * _bce(cls_p, onehot)) / (cnt * _NC)

    zs, acts, ids = [], [], []
    zs.append(sel[:, 4])
    acts.append(jnp.ones((_NT,), jnp.float32))
    ids.append(flat_obj)
    for a in range(_NA):
        zs.append(g[:, a * (_NC + 5) + 4])
        acts.append((ious[a] > _IGNORE).astype(jnp.float32))
        ids.append(((b * _NA + a) * _NG + gj) * _NG + gi)
    zcat = jnp.concatenate(zs)
    acat = jnp.concatenate(acts) > 0.5
    icat = jnp.concatenate(ids)
    n4 = 4 * _NT
    slot = lax.broadcasted_iota(jnp.int32, (n4,), 0)
    key = jnp.where(acat, icat, -1 - slot)
    r4 = lax.broadcasted_iota(jnp.int32, (n4, n4), 0)
    c4 = lax.broadcasted_iota(jnp.int32, (n4, n4), 1)
    eq4 = key[:, None] == key[None, :]
    first = ~jnp.any(eq4 & (c4 < r4), axis=1)
    use = (acat & first).astype(f32)
    nz = jnp.sum(use)
    ncnt = _TOTAL_CELLS - nz
    pz = jnp.clip(jax.nn.sigmoid(zcat), 1e-12, 1.0 - 1e-12)
    noobj_sum = s_dense - jnp.sum(use * (-jnp.log(1.0 - pz)))
    loss_conf_noobj = noobj_sum / ncnt
    loss_conf = loss_conf_obj + _NOOBJ_SCALE * loss_conf_noobj

    rank = jnp.sum(
        ((flat_obj[None, :] < flat_obj[:, None]) & winner[None, :]).astype(
            jnp.int32
        ),
        axis=1,
    )
    ridx = lax.broadcasted_iota(jnp.int32, (_NT, _NT), 1)
    oh_r = (ridx == rank[:, None]).astype(f32)
    t3 = jnp.sum(oh_r * t[None, :, 3].reshape(1, _NT), axis=1)
    t4 = jnp.sum(oh_r * t[None, :, 4].reshape(1, _NT), axis=1)
    pwb = jnp.exp(pw) * aw
    phb = jnp.exp(ph) * ah
    sq = (jnp.sqrt(jnp.abs(pwb) + 1e-32) - jnp.sqrt(jnp.abs(t3) + 1e-32)) ** 2
    sq += (jnp.sqrt(jnp.abs(phb) + 1e-32) - jnp.sqrt(jnp.abs(t4) + 1e-32)) ** 2
    wh_loss = jnp.sum(wf * sq) / (2.0 * cnt)

    total = loss_x + loss_y + loss_w + loss_h + loss_conf + loss_cls
    return jnp.stack(
        [
            total,
            loss_x + loss_y,
            wh_loss,
            loss_conf,
            loss_cls,
            loss_conf_obj,
            loss_conf_noobj,
            jnp.float32(0.0),
        ]
    )


def _main_body(conf_ref, g_ref, t_ref, o_ref):
    p = jnp.clip(jax.nn.sigmoid(conf_ref[...]), 1e-12, 1.0 - 1e-12)
    s_dense = jnp.sum(-jnp.log(1.0 - p))
    o_ref[...] = _losses(g_ref[...], t_ref[...], s_dense).reshape(1, 8)


@functools.cache
def _sc_gather():
    return functools.partial(
        pl.kernel,
        mesh=plsc.VectorSubcoreMesh(core_axis_name="c", subcore_axis_name="s"),
        out_type=(
            jax.ShapeDtypeStruct((_NT * _CPAD,), jnp.float32),
            jax.ShapeDtypeStruct((_NB * _NA * _SPATIAL,), jnp.float32),
        ),
        scratch_types=[
            pltpu.VMEM((_PER_W,), jnp.int32),
            pltpu.VMEM((_PER_W,), jnp.float32),
            pltpu.VMEM((_HALF,), jnp.float32),
            pltpu.SemaphoreType.DMA,
        ],
    )(_sc_gather_body)


@jax.jit
def kernel(x, targets):
    idx = pl.pallas_call(
        _idx_body,
        out_shape=jax.ShapeDtypeStruct((_NT, _CPAD), jnp.int32),
    )(targets)
    g, conf = _sc_gather()(x.reshape(-1), idx.reshape(-1))
    out = pl.pallas_call(
        _main_body,
        out_shape=jax.ShapeDtypeStruct((1, 8), jnp.float32),
    )(conf.reshape(-1, 128), g.reshape(_NT, _CPAD), targets)
    return tuple(out[0, i] for i in range(7))


# speedup vs baseline: 2.0952x; 2.0952x over previous
import functools

import jax
import jax.numpy as jnp
import numpy as np
from jax import lax
from jax.experimental import pallas as pl
from jax.experimental.pallas import tpu as pltpu
from jax.experimental.pallas import tpu_sc as plsc

_NB, _NA, _NG, _NC = 16, 3, 52, 80
_NCH = _NA * (_NC + 5)
_SPATIAL = _NG * _NG
_TOTAL_CELLS = float(_NB * _NA * _SPATIAL)
_NT = 64
_CPAD = 256
_LANE = 128
_NW = 32
_PER_W = _NT * _CPAD // _NW
_AW = np.array([10.0, 16.0, 33.0], np.float32) / 8.0
_AH = np.array([13.0, 30.0, 23.0], np.float32) / 8.0
_IGNORE = 0.5
_NOOBJ_SCALE = 100.0


def _bce(p, t):
    p = jnp.clip(p, 1e-12, 1.0 - 1e-12)
    return -(t * jnp.log(p) + (1.0 - t) * jnp.log(1.0 - p))


def _target_geometry(t):
    b = t[:, 0].astype(jnp.int32)
    gx = t[:, 2] * _NG
    gy = t[:, 3] * _NG
    gw = t[:, 4] * _NG
    gh = t[:, 5] * _NG
    gi = gx.astype(jnp.int32)
    gj = gy.astype(jnp.int32)
    ious = []
    for a in range(_NA):
        inter = jnp.minimum(_AW[a], gw) * jnp.minimum(_AH[a], gh)
        union = _AW[a] * _AH[a] + 1e-16 + gw * gh - inter
        ious.append(inter / union)
    bn = jnp.where(
        (ious[0] >= ious[1]) & (ious[0] >= ious[2]),
        0,
        jnp.where(ious[1] >= ious[2], 1, 2),
    ).astype(jnp.int32)
    return b, gi, gj, gx, gy, gw, gh, ious, bn


def _idx_body(t_ref, b_ref, gj_ref, o_ref):
    t = t_ref[...]
    b, gi, gj, *_ = _target_geometry(t)
    b_ref[...] = b
    gj_ref[...] = gj
    row = lax.broadcasted_iota(jnp.int32, (_NT, _CPAD), 0)
    c = jnp.minimum(lax.broadcasted_iota(jnp.int32, (_NT, _CPAD), 1), _NCH - 1)
    o_ref[...] = row * (_CPAD * _LANE) + c * _LANE + gi[:, None]


def _stage_body(b_ref, gj_ref, x1_ref, x2_ref, st_ref, s_ref, acc_ref):
    t = pl.program_id(0)

    @pl.when(t == 0)
    def _init():
        acc_ref[0] = 0.0

    @pl.when(t < _NB * _NA)
    def _dense():
        p = jnp.clip(jax.nn.sigmoid(x2_ref[0, 0]), 1e-12, 1.0 - 1e-12)
        acc_ref[0] += jnp.sum(-jnp.log(1.0 - p))

    r = gj_ref[t] % 8
    v = x1_ref[0, :, pl.ds(r, 1), :].reshape(_NCH, _NG)
    v = jnp.concatenate([v, jnp.zeros((_NCH, _LANE - _NG), jnp.float32)], 1)
    v = jnp.concatenate([v, jnp.zeros((1, _LANE), jnp.float32)], 0)
    st_ref[...] = v

    @pl.when(t == _NT - 1)
    def _emit():
        s_ref[...] = jnp.full((1, 1), acc_ref[0], jnp.float32)


def _conf_index_map(t, b_ref, gj_ref):
    ic = jnp.minimum(t, _NB * _NA - 1)
    return (ic // _NA, 4 + (_NC + 5) * (ic % _NA), 0, 0)


def _sc_gather_body(st_hbm, idx_hbm, out_hbm, idx_v, rows_v, sem):
    wid = lax.axis_index("s") * 2 + lax.axis_index("c")
    base = wid * _PER_W
    pltpu.sync_copy(idx_hbm.at[pl.ds(base, _PER_W)], idx_v)
    pltpu.async_copy(st_hbm.at[idx_v], rows_v, sem).wait()
    pltpu.sync_copy(rows_v, out_hbm.at[pl.ds(base, _PER_W)])


def _losses(g, t, s_dense):
    f32 = jnp.float32
    b, gi, gj, gx, gy, gw, gh, ious, bn = _target_geometry(t)
    labels = t[:, 1].astype(jnp.int32)
    flat_obj = ((b * _NA + bn) * _NG + gj) * _NG + gi

    tidx = lax.broadcasted_iota(jnp.int32, (_NT, _NT), 0)
    oidx = lax.broadcasted_iota(jnp.int32, (_NT, _NT), 1)
    eq = flat_obj[:, None] == flat_obj[None, :]
    winner = ~jnp.any(eq & (oidx > tidx), axis=1)
    wf = winner.astype(f32)
    cnt = jnp.sum(wf)

    b0 = g[:, 0 : _NC + 5]
    b1 = g[:, _NC + 5 : 2 * (_NC + 5)]
    b2 = g[:, 2 * (_NC + 5) : 3 * (_NC + 5)]
    bnf = bn[:, None]
    sel = jnp.where(bnf == 0, b0, jnp.where(bnf == 1, b1, b2))
    px = jax.nn.sigmoid(sel[:, 0])
    py = jax.nn.sigmoid(sel[:, 1])
    pw = sel[:, 2]
    ph = sel[:, 3]
    pconf = jax.nn.sigmoid(sel[:, 4])
    aw = jnp.where(bn == 0, _AW[0], jnp.where(bn == 1, _AW[1], _AW[2]))
    ah = jnp.where(bn == 0, _AH[0], jnp.where(bn == 1, _AH[1], _AH[2]))

    tx = gx - jnp.floor(gx)
    ty = gy - jnp.floor(gy)
    tw = jnp.log(gw / aw + 1e-16)
    th = jnp.log(gh / ah + 1e-16)
    loss_x = jnp.sum(wf * (px - tx) ** 2) / cnt
    loss_y = jnp.sum(wf * (py - ty) ** 2) / cnt
    loss_w = jnp.sum(wf * (pw - tw) ** 2) / cnt
    loss_h = jnp.sum(wf * (ph - th) ** 2) / cnt
    loss_conf_obj = jnp.sum(wf * _bce(pconf, 1.0)) / cnt

    cls_p = jax.nn.sigmoid(sel[:, 5:])
    cidx = lax.broadcasted_iota(jnp.int32, (_NT, _NC), 1)
    onehot = (cidx == labels[:, None]).astype(f32)
    loss_cls = jnp.sum(wf[:, None] * _bce(cls_p, onehot)) / (cnt * _NC)

    zs, acts, ids = [], [], []
    zs.append(sel[:, 4])
    acts.append(jnp.ones((_NT,), jnp.float32))
    ids.append(flat_obj)
    for a in range(_NA):
        zs.append(g[:, a * (_NC + 5) + 4])
        acts.append((ious[a] > _IGNORE).astype(jnp.float32))
        ids.append(((b * _NA + a) * _NG + gj) * _NG + gi)
    zcat = jnp.concatenate(zs)
    acat = jnp.concatenate(acts) > 0.5
    icat = jnp.concatenate(ids)
    n4 = 4 * _NT
    slot = lax.broadcasted_iota(jnp.int32, (n4,), 0)
    key = jnp.where(acat, icat, -1 - slot)
    r4 = lax.broadcasted_iota(jnp.int32, (n4, n4), 0)
    c4 = lax.broadcasted_iota(jnp.int32, (n4, n4), 1)
    eq4 = key[:, None] == key[None, :]
    first = ~jnp.any(eq4 & (c4 < r4), axis=1)
    use = (acat & first).astype(f32)
    nz = jnp.sum(use)
    ncnt = _TOTAL_CELLS - nz
    pz = jnp.clip(jax.nn.sigmoid(zcat), 1e-12, 1.0 - 1e-12)
    noobj_sum = s_dense - jnp.sum(use * (-jnp.log(1.0 - pz)))
    loss_conf_noobj = noobj_sum / ncnt
    loss_conf = loss_conf_obj + _NOOBJ_SCALE * loss_conf_noobj

    rank = jnp.sum(
        ((flat_obj[None, :] < flat_obj[:, None]) & winner[None, :]).astype(
            jnp.int32
        ),
        axis=1,
    )
    ridx = lax.broadcasted_iota(jnp.int32, (_NT, _NT), 1)
    oh_r = (ridx == rank[:, None]).astype(f32)
    t3 = jnp.sum(oh_r * t[None, :, 3].reshape(1, _NT), axis=1)
    t4 = jnp.sum(oh_r * t[None, :, 4].reshape(1, _NT), axis=1)
    pwb = jnp.exp(pw) * aw
    phb = jnp.exp(ph) * ah
    sq = (jnp.sqrt(jnp.abs(pwb) + 1e-32) - jnp.sqrt(jnp.abs(t3) + 1e-32)) ** 2
    sq += (jnp.sqrt(jnp.abs(phb) + 1e-32) - jnp.sqrt(jnp.abs(t4) + 1e-32)) ** 2
    wh_loss = jnp.sum(wf * sq) / (2.0 * cnt)

    total = loss_x + loss_y + loss_w + loss_h + loss_conf + loss_cls
    return jnp.stack(
        [
            total,
            loss_x + loss_y,
            wh_loss,
            loss_conf,
            loss_cls,
            loss_conf_obj,
            loss_conf_noobj,
            jnp.float32(0.0),
        ]
    )


def _main_body(g_ref, t_ref, s_ref, o_ref):
    o_ref[...] = _losses(g_ref[...], t_ref[...], s_ref[0, 0]).reshape(1, 8)


@functools.cache
def _sc_gather():
    return functools.partial(
        pl.kernel,
        mesh=plsc.VectorSubcoreMesh(core_axis_name="c", subcore_axis_name="s"),
        out_type=jax.ShapeDtypeStruct((_NT * _CPAD,), jnp.float32),
        scratch_types=[
            pltpu.VMEM((_PER_W,), jnp.int32),
            pltpu.VMEM((_PER_W,), jnp.float32),
            pltpu.SemaphoreType.DMA,
        ],
    )(_sc_gather_body)


@jax.jit
def kernel(x, targets):
    bv, gjv, idx = pl.pallas_call(
        _idx_body,
        out_shape=(
            jax.ShapeDtypeStruct((_NT,), jnp.int32),
            jax.ShapeDtypeStruct((_NT,), jnp.int32),
            jax.ShapeDtypeStruct((_NT, _CPAD), jnp.int32),
        ),
    )(targets)
    staged, ssum = pl.pallas_call(
        _stage_body,
        grid_spec=pltpu.PrefetchScalarGridSpec(
            num_scalar_prefetch=2,
            grid=(_NT,),
            in_specs=[
                pl.BlockSpec(
                    (1, _NCH, 8, _NG),
                    lambda t, b_ref, gj_ref: (b_ref[t], 0, gj_ref[t] // 8, 0),
                ),
                pl.BlockSpec((1, 1, _NG, _NG), _conf_index_map),
            ],
            out_specs=[
                pl.BlockSpec((_CPAD, _LANE), lambda t, b_ref, gj_ref: (t, 0)),
                pl.BlockSpec((1, 1), lambda t, b_ref, gj_ref: (0, 0)),
            ],
            scratch_shapes=[pltpu.SMEM((1,), jnp.float32)],
        ),
        out_shape=(
            jax.ShapeDtypeStruct((_NT * _CPAD, _LANE), jnp.float32),
            jax.ShapeDtypeStruct((1, 1), jnp.float32),
        ),
    )(bv, gjv, x, x)
    g = _sc_gather()(staged.reshape(-1), idx.reshape(-1))
    out = pl.pallas_call(
        _main_body,
        out_shape=jax.ShapeDtypeStruct((1, 8), jnp.float32),
    )(g.reshape(_NT, _CPAD), targets, ssum)
    return tuple(out[0, i] for i in range(7))


# speedup vs baseline: 2.2219x; 1.0605x over previous
import functools

import jax
import jax.numpy as jnp
import numpy as np
from jax import lax
from jax.experimental import pallas as pl
from jax.experimental.pallas import tpu as pltpu
from jax.experimental.pallas import tpu_sc as plsc

_NB, _NA, _NG, _NC = 16, 3, 52, 80
_NCH = _NA * (_NC + 5)
_SPATIAL = _NG * _NG
_TOTAL_CELLS = float(_NB * _NA * _SPATIAL)
_NT = 64
_RPT = 88
_GPT = 96
_LANE = 128
_NW = 32
_PER_W = _NT * _GPT // _NW
_AW = np.array([10.0, 16.0, 33.0], np.float32) / 8.0
_AH = np.array([13.0, 30.0, 23.0], np.float32) / 8.0
_IGNORE = 0.5
_NOOBJ_SCALE = 100.0


def _bce(p, t):
    p = jnp.clip(p, 1e-12, 1.0 - 1e-12)
    return -(t * jnp.log(p) + (1.0 - t) * jnp.log(1.0 - p))


def _target_geometry(t):
    b = t[:, 0].astype(jnp.int32)
    gx = t[:, 2] * _NG
    gy = t[:, 3] * _NG
    gw = t[:, 4] * _NG
    gh = t[:, 5] * _NG
    gi = gx.astype(jnp.int32)
    gj = gy.astype(jnp.int32)
    ious = []
    for a in range(_NA):
        inter = jnp.minimum(_AW[a], gw) * jnp.minimum(_AH[a], gh)
        union = _AW[a] * _AH[a] + 1e-16 + gw * gh - inter
        ious.append(inter / union)
    bn = jnp.where(
        (ious[0] >= ious[1]) & (ious[0] >= ious[2]),
        0,
        jnp.where(ious[1] >= ious[2], 1, 2),
    ).astype(jnp.int32)
    return b, gi, gj, gx, gy, gw, gh, ious, bn


def _idx_body(t_ref, b_ref, gj_ref, a_ref, o_ref):
    t = t_ref[...]
    b, gi, gj, gx, gy, gw, gh, ious, bn = _target_geometry(t)
    b_ref[...] = b
    gj_ref[...] = gj
    a_ref[...] = bn
    row = lax.broadcasted_iota(jnp.int32, (_NT, _GPT), 0)
    c = lax.broadcasted_iota(jnp.int32, (_NT, _GPT), 1)
    r = jnp.where(c < _RPT - 1, c, 0)
    o_ref[...] = row * (_RPT * _LANE) + r * _LANE + gi[:, None]


def _stage_body(b_ref, gj_ref, a_ref, x1_ref, xc1_ref, xc2_ref, x2_ref,
                st_ref, s_ref, acc_ref):
    t = pl.program_id(0)

    @pl.when(t == 0)
    def _init():
        acc_ref[0] = 0.0

    @pl.when(t < _NB * _NA)
    def _dense():
        p = jnp.clip(jax.nn.sigmoid(x2_ref[0, 0]), 1e-12, 1.0 - 1e-12)
        acc_ref[0] += jnp.sum(-jnp.log(1.0 - p))

    r = gj_ref[t] % 8
    v = x1_ref[0, :, pl.ds(r, 1), :].reshape(_NC + 5, _NG)
    c1 = xc1_ref[0, :, pl.ds(r, 1), :].reshape(1, _NG)
    c2 = xc2_ref[0, :, pl.ds(r, 1), :].reshape(1, _NG)
    m = jnp.concatenate([v, c1, c2, jnp.zeros((1, _NG), jnp.float32)], 0)
    m = jnp.concatenate([m, jnp.zeros((_RPT, _LANE - _NG), jnp.float32)], 1)
    st_ref[...] = m

    @pl.when(t == _NT - 1)
    def _emit():
        s_ref[...] = jnp.full((1, 1), acc_ref[0], jnp.float32)


def _conf_index_map(t, b_ref, gj_ref, a_ref):
    ic = jnp.minimum(t, _NB * _NA - 1)
    return (ic // _NA, 4 + (_NC + 5) * (ic % _NA), 0, 0)


def _sc_gather_body(st_hbm, idx_hbm, out_hbm, idx_v, rows_v, sem):
    wid = lax.axis_index("s") * 2 + lax.axis_index("c")
    base = wid * _PER_W
    pltpu.sync_copy(idx_hbm.at[pl.ds(base, _PER_W)], idx_v)
    pltpu.async_copy(st_hbm.at[idx_v], rows_v, sem).wait()
    pltpu.sync_copy(rows_v, out_hbm.at[pl.ds(base, _PER_W)])


def _losses(g, t, s_dense):
    f32 = jnp.float32
    b, gi, gj, gx, gy, gw, gh, ious, bn = _target_geometry(t)
    labels = t[:, 1].astype(jnp.int32)
    flat_obj = ((b * _NA + bn) * _NG + gj) * _NG + gi

    tidx = lax.broadcasted_iota(jnp.int32, (_NT, _NT), 0)
    oidx = lax.broadcasted_iota(jnp.int32, (_NT, _NT), 1)
    eq = flat_obj[:, None] == flat_obj[None, :]
    winner = ~jnp.any(eq & (oidx > tidx), axis=1)
    wf = winner.astype(f32)
    cnt = jnp.sum(wf)

    sel = g[:, 0 : _NC + 5]
    px = jax.nn.sigmoid(sel[:, 0])
    py = jax.nn.sigmoid(sel[:, 1])
    pw = sel[:, 2]
    ph = sel[:, 3]
    pconf = jax.nn.sigmoid(sel[:, 4])
    aw = jnp.where(bn == 0, _AW[0], jnp.where(bn == 1, _AW[1], _AW[2]))
    ah = jnp.where(bn == 0, _AH[0], jnp.where(bn == 1, _AH[1], _AH[2]))

    tx = gx - jnp.floor(gx)
    ty = gy - jnp.floor(gy)
    tw = jnp.log(gw / aw + 1e-16)
    th = jnp.log(gh / ah + 1e-16)
    loss_x = jnp.sum(wf * (px - tx) ** 2) / cnt
    loss_y = jnp.sum(wf * (py - ty) ** 2) / cnt
    loss_w = jnp.sum(wf * (pw - tw) ** 2) / cnt
    loss_h = jnp.sum(wf * (ph - th) ** 2) / cnt
    loss_conf_obj = jnp.sum(wf * _bce(pconf, 1.0)) / cnt

    cls_p = jax.nn.sigmoid(sel[:, 5:])
    cidx = lax.broadcasted_iota(jnp.int32, (_NT, _NC), 1)
    onehot = (cidx == labels[:, None]).astype(f32)
    loss_cls = jnp.sum(wf[:, None] * _bce(cls_p, onehot)) / (cnt * _NC)

    zs, acts, ids = [], [], []
    zs.append(sel[:, 4])
    acts.append(jnp.ones((_NT,), jnp.float32))
    ids.append(flat_obj)
    for a in range(_NA):
        d = (a - bn) % _NA
        za = jnp.where(
            d == 0, sel[:, 4], jnp.where(d == 1, g[:, _NC + 5], g[:, _NC + 6])
        )
        zs.append(za)
        acts.append((ious[a] > _IGNORE).astype(jnp.float32))
        ids.append(((b * _NA + a) * _NG + gj) * _NG + gi)
    zcat = jnp.concatenate(zs)
    acat = jnp.concatenate(acts) > 0.5
    icat = jnp.concatenate(ids)
    n4 = 4 * _NT
    slot = lax.broadcasted_iota(jnp.int32, (n4,), 0)
    key = jnp.where(acat, icat, -1 - slot)
    r4 = lax.broadcasted_iota(jnp.int32, (n4, n4), 0)
    c4 = lax.broadcasted_iota(jnp.int32, (n4, n4), 1)
    eq4 = key[:, None] == key[None, :]
    first = ~jnp.any(eq4 & (c4 < r4), axis=1)
    use = (acat & first).astype(f32)
    nz = jnp.sum(use)
    ncnt = _TOTAL_CELLS - nz
    pz = jnp.clip(jax.nn.sigmoid(zcat), 1e-12, 1.0 - 1e-12)
    noobj_sum = s_dense - jnp.sum(use * (-jnp.log(1.0 - pz)))
    loss_conf_noobj = noobj_sum / ncnt
    loss_conf = loss_conf_obj + _NOOBJ_SCALE * loss_conf_noobj

    rank = jnp.sum(
        ((flat_obj[None, :] < flat_obj[:, None]) & winner[None, :]).astype(
            jnp.int32
        ),
        axis=1,
    )
    ridx = lax.broadcasted_iota(jnp.int32, (_NT, _NT), 1)
    oh_r = (ridx == rank[:, None]).astype(f32)
    t3 = jnp.sum(oh_r * t[None, :, 3].reshape(1, _NT), axis=1)
    t4 = jnp.sum(oh_r * t[None, :, 4].reshape(1, _NT), axis=1)
    pwb = jnp.exp(pw) * aw
    phb = jnp.exp(ph) * ah
    sq = (jnp.sqrt(jnp.abs(pwb) + 1e-32) - jnp.sqrt(jnp.abs(t3) + 1e-32)) ** 2
    sq += (jnp.sqrt(jnp.abs(phb) + 1e-32) - jnp.sqrt(jnp.abs(t4) + 1e-32)) ** 2
    wh_loss = jnp.sum(wf * sq) / (2.0 * cnt)

    total = loss_x + loss_y + loss_w + loss_h + loss_conf + loss_cls
    return jnp.stack(
        [
            total,
            loss_x + loss_y,
            wh_loss,
            loss_conf,
            loss_cls,
            loss_conf_obj,
            loss_conf_noobj,
            jnp.float32(0.0),
        ]
    )


def _main_body(g_ref, t_ref, s_ref, o_ref):
    o_ref[...] = _losses(g_ref[...], t_ref[...], s_ref[0, 0]).reshape(1, 8)


@functools.cache
def _sc_gather():
    return functools.partial(
        pl.kernel,
        mesh=plsc.VectorSubcoreMesh(core_axis_name="c", subcore_axis_name="s"),
        out_type=jax.ShapeDtypeStruct((_NT * _GPT,), jnp.float32),
        scratch_types=[
            pltpu.VMEM((_PER_W,), jnp.int32),
            pltpu.VMEM((_PER_W,), jnp.float32),
            pltpu.SemaphoreType.DMA,
        ],
    )(_sc_gather_body)


@jax.jit
def kernel(x, targets):
    bv, gjv, av, idx = pl.pallas_call(
        _idx_body,
        out_shape=(
            jax.ShapeDtypeStruct((_NT,), jnp.int32),
            jax.ShapeDtypeStruct((_NT,), jnp.int32),
            jax.ShapeDtypeStruct((_NT,), jnp.int32),
            jax.ShapeDtypeStruct((_NT, _GPT), jnp.int32),
        ),
    )(targets)
    staged, ssum = pl.pallas_call(
        _stage_body,
        grid_spec=pltpu.PrefetchScalarGridSpec(
            num_scalar_prefetch=3,
            grid=(_NT,),
            in_specs=[
                pl.BlockSpec(
                    (1, _NC + 5, 8, _NG),
                    lambda t, b, gj, a: (b[t], a[t], gj[t] // 8, 0),
                ),
                pl.BlockSpec(
                    (1, 1, 8, _NG),
                    lambda t, b, gj, a: (
                        b[t],
                        4 + (_NC + 5) * ((a[t] + 1) % _NA),
                        gj[t] // 8,
                        0,
                    ),
                ),
                pl.BlockSpec(
                    (1, 1, 8, _NG),
                    lambda t, b, gj, a: (
                        b[t],
                        4 + (_NC + 5) * ((a[t] + 2) % _NA),
                        gj[t] // 8,
                        0,
                    ),
                ),
                pl.BlockSpec((1, 1, _NG, _NG), _conf_index_map),
            ],
            out_specs=[
                pl.BlockSpec((_RPT, _LANE), lambda t, b, gj, a: (t, 0)),
                pl.BlockSpec((1, 1), lambda t, b, gj, a: (0, 0)),
            ],
            scratch_shapes=[pltpu.SMEM((1,), jnp.float32)],
        ),
        out_shape=(
            jax.ShapeDtypeStruct((_NT * _RPT, _LANE), jnp.float32),
            jax.ShapeDtypeStruct((1, 1), jnp.float32),
        ),
    )(bv, gjv, av, x, x, x, x)
    g = _sc_gather()(staged.reshape(-1), idx.reshape(-1))
    out = pl.pallas_call(
        _main_body,
        out_shape=jax.ShapeDtypeStruct((1, 8), jnp.float32),
    )(g.reshape(_NT, _GPT), targets, ssum)
    return tuple(out[0, i] for i in range(7))
